# baseline (device time: 28593 ns/iter reference)
import jax
import jax.numpy as jnp
from jax import lax
from jax.experimental import pallas as pl
from jax.experimental.pallas import tpu as pltpu

N_DEV = 4
N_LAYERS = 3
B = 512
D = 256
N_CH = 8
ROWS = B // N_CH
B_OUT = B // N_DEV
HALF = N_CH // 2
ORDER = tuple(q for pair in zip(range(HALF), range(HALF, N_CH)) for q in pair)


def kernel(x, Win0, Wout0, Win1, Wout1, Win2, Wout2):
    def body(x_ref, win0_ref, wout0_ref, win1_ref, wout1_ref,
             win2_ref, wout2_ref, out_ref,
             part_ref, r1_ref, s_ref, r2_ref, xn_ref,
             ph1_send, ph1_recv, ph2_send, ph2_recv):
        my = lax.axis_index("i")
        py = my + 1 - 2 * lax.rem(my, 2)
        px = 3 - my

        barrier_sem = pltpu.get_barrier_semaphore()
        for nbr in (py, px):
            pl.semaphore_signal(
                barrier_sem, inc=1,
                device_id=(nbr,), device_id_type=pl.DeviceIdType.MESH,
            )
        pl.semaphore_wait(barrier_sem, 2)

        win_refs = [win0_ref, win1_ref, win2_ref]
        wout_refs = [wout0_ref, wout1_ref, wout2_ref]

        def p1_partner(q):
            return px if q < HALF else py

        def p2_partner(q):
            return py if q < HALF else px

        def block_out(xq, wi, wo):
            h = jnp.maximum(
                jax.lax.dot_general(
                    xq, wi, (((1,), (0,)), ((), ())),
                    preferred_element_type=jnp.float32,
                ),
                0.0,
            )
            return jax.lax.dot_general(
                h.astype(jnp.bfloat16), wo, (((1,), (0,)), ((), ())),
                preferred_element_type=jnp.float32,
            )

        d1 = {}
        d2 = {}
        pvals = {}
        all_descs = []

        def fire_ph1(L, q, pq):
            pvals[(L, q)] = pq
            part_ref[L, q] = pq.astype(jnp.bfloat16)
            d = pltpu.make_async_remote_copy(
                src_ref=part_ref.at[L, q],
                dst_ref=r1_ref.at[L, q],
                send_sem=ph1_send.at[L, q],
                recv_sem=ph1_recv.at[L, q],
                device_id=(p1_partner(q),),
                device_id_type=pl.DeviceIdType.MESH,
            )
            d.start()
            d1[(L, q)] = d
            all_descs.append(d)

        wi = win_refs[0][...].astype(jnp.bfloat16)
        wo = wout_refs[0][...].astype(jnp.bfloat16)
        for q in ORDER:
            xq = x_ref[pl.ds(q * ROWS, ROWS), :].astype(jnp.bfloat16)
            fire_ph1(0, q, block_out(xq, wi, wo))

        for L in range(N_LAYERS):
            for q in ORDER:
                d1[(L, q)].wait_recv()
                s = pvals[(L, q)] + r1_ref[L, q].astype(jnp.float32)
                s_ref[L, q] = s.astype(jnp.bfloat16)
                d = pltpu.make_async_remote_copy(
                    src_ref=s_ref.at[L, q],
                    dst_ref=r2_ref.at[L, q],
                    send_sem=ph2_send.at[L, q],
                    recv_sem=ph2_recv.at[L, q],
                    device_id=(p2_partner(q),),
                    device_id_type=pl.DeviceIdType.MESH,
                )
                d.start()
                d2[(L, q)] = d
                all_descs.append(d)

            if L < N_LAYERS - 1:
                wi = win_refs[L + 1][...].astype(jnp.bfloat16)
                wo = wout_refs[L + 1][...].astype(jnp.bfloat16)
                for q in ORDER:
                    d2[(L, q)].wait_recv()
                    f = (s_ref[L, q].astype(jnp.float32) +
                         r2_ref[L, q].astype(jnp.float32)).astype(jnp.bfloat16)
                    fire_ph1(L + 1, q, block_out(f, wi, wo))
            else:
                for q in ORDER:
                    d2[(L, q)].wait_recv()
                    xn_ref[pl.ds(q * ROWS, ROWS), :] = \
                        s_ref[L, q].astype(jnp.float32) + \
                        r2_ref[L, q].astype(jnp.float32)
                out_ref[...] = xn_ref[pl.ds(my * B_OUT, B_OUT), :]

        for d in all_descs:
            d.wait_send()

    return pl.pallas_call(
        body,
        out_shape=jax.ShapeDtypeStruct((B_OUT, D), jnp.float32),
        in_specs=[pl.BlockSpec(memory_space=pltpu.VMEM)] * 7,
        out_specs=pl.BlockSpec(memory_space=pltpu.VMEM),
        scratch_shapes=[
            pltpu.VMEM((N_LAYERS, N_CH, ROWS, D), jnp.bfloat16),
            pltpu.VMEM((N_LAYERS, N_CH, ROWS, D), jnp.bfloat16),
            pltpu.VMEM((N_LAYERS, N_CH, ROWS, D), jnp.bfloat16),
            pltpu.VMEM((N_LAYERS, N_CH, ROWS, D), jnp.bfloat16),
            pltpu.VMEM((B, D), jnp.float32),
            pltpu.SemaphoreType.DMA((N_LAYERS, N_CH)),
            pltpu.SemaphoreType.DMA((N_LAYERS, N_CH)),
            pltpu.SemaphoreType.DMA((N_LAYERS, N_CH)),
            pltpu.SemaphoreType.DMA((N_LAYERS, N_CH)),
        ],
        compiler_params=pltpu.CompilerParams(collective_id=0),
    )(x, Win0, Wout0, Win1, Wout1, Win2, Wout2)


# device time: 28263 ns/iter; 1.0117x vs baseline; 1.0117x over previous
import jax
import jax.numpy as jnp
from jax import lax
from jax.experimental import pallas as pl
from jax.experimental.pallas import tpu as pltpu

N_DEV = 4
N_LAYERS = 3
B = 512
D = 256
N_CH = 4
ROWS = B // N_CH
B_OUT = B // N_DEV
HALF = N_CH // 2
ORDER = tuple(q for pair in zip(range(HALF), range(HALF, N_CH)) for q in pair)


def kernel(x, Win0, Wout0, Win1, Wout1, Win2, Wout2):
    def body(x_ref, win0_ref, wout0_ref, win1_ref, wout1_ref,
             win2_ref, wout2_ref, out_ref,
             part_ref, r1_ref, s_ref, r2_ref, xn_ref, wv_in, wv_out,
             ph1_send, ph1_recv, ph2_send, ph2_recv, wsems):
        my = lax.axis_index("i")
        py = my + 1 - 2 * lax.rem(my, 2)
        px = 3 - my

        barrier_sem = pltpu.get_barrier_semaphore()
        for nbr in (py, px):
            pl.semaphore_signal(
                barrier_sem, inc=1,
                device_id=(nbr,), device_id_type=pl.DeviceIdType.MESH,
            )
        pl.semaphore_wait(barrier_sem, 2)

        win_hbm = [win0_ref, win1_ref, win2_ref]
        wout_hbm = [wout0_ref, wout1_ref, wout2_ref]
        wcopies = []
        for L in range(N_LAYERS):
            cin = pltpu.make_async_copy(
                win_hbm[L], wv_in.at[L], wsems.at[0, L])
            cout = pltpu.make_async_copy(
                wout_hbm[L], wv_out.at[L], wsems.at[1, L])
            cin.start()
            cout.start()
            wcopies.append((cin, cout))

        def load_weights(L):
            cin, cout = wcopies[L]
            cin.wait()
            cout.wait()
            return (wv_in[L].astype(jnp.bfloat16),
                    wv_out[L].astype(jnp.bfloat16))

        def p1_partner(q):
            return px if q < HALF else py

        def p2_partner(q):
            return py if q < HALF else px

        def block_out(xq, wi, wo):
            h = jnp.maximum(
                jax.lax.dot_general(
                    xq, wi, (((1,), (0,)), ((), ())),
                    preferred_element_type=jnp.float32,
                ),
                0.0,
            )
            return jax.lax.dot_general(
                h.astype(jnp.bfloat16), wo, (((1,), (0,)), ((), ())),
                preferred_element_type=jnp.float32,
            )

        d1 = {}
        d2 = {}
        pvals = {}
        all_descs = []

        def fire_ph1(L, q, pq):
            pvals[(L, q)] = pq
            part_ref[L, q] = pq.astype(jnp.bfloat16)
            d = pltpu.make_async_remote_copy(
                src_ref=part_ref.at[L, q],
                dst_ref=r1_ref.at[L, q],
                send_sem=ph1_send.at[L, q],
                recv_sem=ph1_recv.at[L, q],
                device_id=(p1_partner(q),),
                device_id_type=pl.DeviceIdType.MESH,
            )
            d.start()
            d1[(L, q)] = d
            all_descs.append(d)

        wi, wo = load_weights(0)
        for q in ORDER:
            xq = x_ref[pl.ds(q * ROWS, ROWS), :].astype(jnp.bfloat16)
            fire_ph1(0, q, block_out(xq, wi, wo))

        for L in range(N_LAYERS):
            for q in ORDER:
                d1[(L, q)].wait_recv()
                s = pvals[(L, q)] + r1_ref[L, q].astype(jnp.float32)
                s_ref[L, q] = s.astype(jnp.bfloat16)
                d = pltpu.make_async_remote_copy(
                    src_ref=s_ref.at[L, q],
                    dst_ref=r2_ref.at[L, q],
                    send_sem=ph2_send.at[L, q],
                    recv_sem=ph2_recv.at[L, q],
                    device_id=(p2_partner(q),),
                    device_id_type=pl.DeviceIdType.MESH,
                )
                d.start()
                d2[(L, q)] = d
                all_descs.append(d)

            if L < N_LAYERS - 1:
                wi, wo = load_weights(L + 1)
                for q in ORDER:
                    d2[(L, q)].wait_recv()
                    f = (s_ref[L, q].astype(jnp.float32) +
                         r2_ref[L, q].astype(jnp.float32)).astype(jnp.bfloat16)
                    fire_ph1(L + 1, q, block_out(f, wi, wo))
            else:
                for q in ORDER:
                    d2[(L, q)].wait_recv()
                    xn_ref[pl.ds(q * ROWS, ROWS), :] = \
                        s_ref[L, q].astype(jnp.float32) + \
                        r2_ref[L, q].astype(jnp.float32)
                out_ref[...] = xn_ref[pl.ds(my * B_OUT, B_OUT), :]

        for d in all_descs:
            d.wait_send()

    return pl.pallas_call(
        body,
        out_shape=jax.ShapeDtypeStruct((B_OUT, D), jnp.float32),
        in_specs=[pl.BlockSpec(memory_space=pltpu.VMEM)] +
                 [pl.BlockSpec(memory_space=pl.ANY)] * 6,
        out_specs=pl.BlockSpec(memory_space=pltpu.VMEM),
        scratch_shapes=[
            pltpu.VMEM((N_LAYERS, N_CH, ROWS, D), jnp.bfloat16),
            pltpu.VMEM((N_LAYERS, N_CH, ROWS, D), jnp.bfloat16),
            pltpu.VMEM((N_LAYERS, N_CH, ROWS, D), jnp.bfloat16),
            pltpu.VMEM((N_LAYERS, N_CH, ROWS, D), jnp.bfloat16),
            pltpu.VMEM((B, D), jnp.float32),
            pltpu.VMEM((N_LAYERS, D, 2048 // N_DEV), jnp.float32),
            pltpu.VMEM((N_LAYERS, 2048 // N_DEV, D), jnp.float32),
            pltpu.SemaphoreType.DMA((N_LAYERS, N_CH)),
            pltpu.SemaphoreType.DMA((N_LAYERS, N_CH)),
            pltpu.SemaphoreType.DMA((N_LAYERS, N_CH)),
            pltpu.SemaphoreType.DMA((N_LAYERS, N_CH)),
            pltpu.SemaphoreType.DMA((2, N_LAYERS)),
        ],
        compiler_params=pltpu.CompilerParams(collective_id=0),
    )(x, Win0, Wout0, Win1, Wout1, Win2, Wout2)


# device time: 26901 ns/iter; 1.0629x vs baseline; 1.0506x over previous
import jax
import jax.numpy as jnp
from jax import lax
from jax.experimental import pallas as pl
from jax.experimental.pallas import tpu as pltpu

N_DEV = 4
N_LAYERS = 3
B = 512
D = 256
N_CH = 4
ROWS = B // N_CH
B_OUT = B // N_DEV
HALF = N_CH // 2
ORDER = tuple(q for pair in zip(range(HALF), range(HALF, N_CH)) for q in pair)


def kernel(x, Win0, Wout0, Win1, Wout1, Win2, Wout2):
    def body(x_ref, win0_ref, wout0_ref, win1_ref, wout1_ref,
             win2_ref, wout2_ref, out_ref,
             part_ref, r1_ref, s_ref, r2_ref, p2_ref, rsl_ref,
             wv_in, wv_out,
             ph1_send, ph1_recv, ph2_send, ph2_recv,
             rsl_send, rsl_recv, wsems):
        my = lax.axis_index("i")
        py = my + 1 - 2 * lax.rem(my, 2)
        px = 3 - my

        barrier_sem = pltpu.get_barrier_semaphore()
        for k in range(1, N_DEV):
            pl.semaphore_signal(
                barrier_sem, inc=1,
                device_id=((my + k) % N_DEV,),
                device_id_type=pl.DeviceIdType.MESH,
            )
        pl.semaphore_wait(barrier_sem, N_DEV - 1)

        win_hbm = [win0_ref, win1_ref, win2_ref]
        wout_hbm = [wout0_ref, wout1_ref, wout2_ref]
        wcopies = []
        for L in range(N_LAYERS):
            cin = pltpu.make_async_copy(
                win_hbm[L], wv_in.at[L], wsems.at[0, L])
            cout = pltpu.make_async_copy(
                wout_hbm[L], wv_out.at[L], wsems.at[1, L])
            cin.start()
            cout.start()
            wcopies.append((cin, cout))

        def load_weights(L):
            cin, cout = wcopies[L]
            cin.wait()
            cout.wait()
            return (wv_in[L].astype(jnp.bfloat16),
                    wv_out[L].astype(jnp.bfloat16))

        def p1_partner(q):
            return px if q < HALF else py

        def p2_partner(q):
            return py if q < HALF else px

        def block_out(xq, wi, wo):
            h = jnp.maximum(
                jax.lax.dot_general(
                    xq, wi, (((1,), (0,)), ((), ())),
                    preferred_element_type=jnp.float32,
                ),
                0.0,
            )
            return jax.lax.dot_general(
                h.astype(jnp.bfloat16), wo, (((1,), (0,)), ((), ())),
                preferred_element_type=jnp.float32,
            )

        d1 = {}
        d2 = {}
        pvals = {}
        all_descs = []

        def fire_ph1(L, q, pq):
            pvals[(L, q)] = pq
            part_ref[L, q] = pq.astype(jnp.bfloat16)
            d = pltpu.make_async_remote_copy(
                src_ref=part_ref.at[L, q],
                dst_ref=r1_ref.at[L, q],
                send_sem=ph1_send.at[L, q],
                recv_sem=ph1_recv.at[L, q],
                device_id=(p1_partner(q),),
                device_id_type=pl.DeviceIdType.MESH,
            )
            d.start()
            d1[(L, q)] = d
            all_descs.append(d)

        wi, wo = load_weights(0)
        for q in ORDER:
            xq = x_ref[pl.ds(q * ROWS, ROWS), :].astype(jnp.bfloat16)
            fire_ph1(0, q, block_out(xq, wi, wo))

        for L in range(N_LAYERS - 1):
            for q in ORDER:
                d1[(L, q)].wait_recv()
                s = pvals[(L, q)] + r1_ref[L, q].astype(jnp.float32)
                s_ref[L, q] = s.astype(jnp.bfloat16)
                d = pltpu.make_async_remote_copy(
                    src_ref=s_ref.at[L, q],
                    dst_ref=r2_ref.at[L, q],
                    send_sem=ph2_send.at[L, q],
                    recv_sem=ph2_recv.at[L, q],
                    device_id=(p2_partner(q),),
                    device_id_type=pl.DeviceIdType.MESH,
                )
                d.start()
                d2[(L, q)] = d
                all_descs.append(d)

            wi, wo = load_weights(L + 1)
            if L < N_LAYERS - 2:
                for q in ORDER:
                    d2[(L, q)].wait_recv()
                    f = (s_ref[L, q].astype(jnp.float32) +
                         r2_ref[L, q].astype(jnp.float32)).astype(jnp.bfloat16)
                    fire_ph1(L + 1, q, block_out(f, wi, wo))
            else:
                for q in ORDER:
                    d2[(L, q)].wait_recv()
                    f = (s_ref[L, q].astype(jnp.float32) +
                         r2_ref[L, q].astype(jnp.float32)).astype(jnp.bfloat16)
                    p2_ref[pl.ds(q * ROWS, ROWS), :] = \
                        block_out(f, wi, wo).astype(jnp.bfloat16)
                rs_descs = []
                for k in (2, 1, 3):
                    j = (my + k) % N_DEV
                    d = pltpu.make_async_remote_copy(
                        src_ref=p2_ref.at[pl.ds(j * B_OUT, B_OUT), :],
                        dst_ref=rsl_ref.at[k - 1],
                        send_sem=rsl_send.at[k - 1],
                        recv_sem=rsl_recv.at[k - 1],
                        device_id=(j,),
                        device_id_type=pl.DeviceIdType.MESH,
                    )
                    d.start()
                    rs_descs.append(d)
                    all_descs.append(d)
                acc = p2_ref[pl.ds(my * B_OUT, B_OUT), :].astype(jnp.float32)
                for d in rs_descs:
                    d.wait_recv()
                for i in range(N_DEV - 1):
                    acc = acc + rsl_ref[i].astype(jnp.float32)
                out_ref[...] = acc

        for d in all_descs:
            d.wait_send()

    return pl.pallas_call(
        body,
        out_shape=jax.ShapeDtypeStruct((B_OUT, D), jnp.float32),
        in_specs=[pl.BlockSpec(memory_space=pltpu.VMEM)] +
                 [pl.BlockSpec(memory_space=pl.ANY)] * 6,
        out_specs=pl.BlockSpec(memory_space=pltpu.VMEM),
        scratch_shapes=[
            pltpu.VMEM((N_LAYERS - 1, N_CH, ROWS, D), jnp.bfloat16),
            pltpu.VMEM((N_LAYERS - 1, N_CH, ROWS, D), jnp.bfloat16),
            pltpu.VMEM((N_LAYERS - 1, N_CH, ROWS, D), jnp.bfloat16),
            pltpu.VMEM((N_LAYERS - 1, N_CH, ROWS, D), jnp.bfloat16),
            pltpu.VMEM((B, D), jnp.bfloat16),
            pltpu.VMEM((N_DEV - 1, B_OUT, D), jnp.bfloat16),
            pltpu.VMEM((N_LAYERS, D, 2048 // N_DEV), jnp.float32),
            pltpu.VMEM((N_LAYERS, 2048 // N_DEV, D), jnp.float32),
            pltpu.SemaphoreType.DMA((N_LAYERS - 1, N_CH)),
            pltpu.SemaphoreType.DMA((N_LAYERS - 1, N_CH)),
            pltpu.SemaphoreType.DMA((N_LAYERS - 1, N_CH)),
            pltpu.SemaphoreType.DMA((N_LAYERS - 1, N_CH)),
            pltpu.SemaphoreType.DMA((N_DEV - 1,)),
            pltpu.SemaphoreType.DMA((N_DEV - 1,)),
            pltpu.SemaphoreType.DMA((2, N_LAYERS)),
        ],
        compiler_params=pltpu.CompilerParams(collective_id=0),
    )(x, Win0, Wout0, Win1, Wout1, Win2, Wout2)
